# baseline (speedup 1.0000x reference)
import functools

import jax
import jax.numpy as jnp
from jax import lax
from jax.experimental import pallas as pl
from jax.experimental.pallas import tpu as pltpu
from jax.experimental.pallas import tpu_sc as plsc

S0 = 32
S1 = 8
NH = 64
NSPLIT = 8


def _gather_rows(emb, x):
    V, D = emb.shape
    B = x.shape[0]
    NC, NS = 2, 16
    NW = NC * NS
    bpw = B // NW

    mesh = plsc.VectorSubcoreMesh(core_axis_name="c", subcore_axis_name="s")

    @functools.partial(
        pl.kernel,
        mesh=mesh,
        out_type=jax.ShapeDtypeStruct((B, D), jnp.float32),
        scratch_types=[
            pltpu.VMEM((bpw,), jnp.int32),
            pltpu.VMEM((bpw, D), jnp.float32),
            pltpu.SemaphoreType.DMA,
            pltpu.SemaphoreType.DMA,
            pltpu.SemaphoreType.DMA,
        ],
    )
    def k(table_hbm, idx_hbm, out_hbm, idx_v, rows_v, sem_a, sem_b, sem_w):
        wid = lax.axis_index("s") * NC + lax.axis_index("c")
        base = wid * bpw
        pltpu.sync_copy(idx_hbm.at[pl.ds(base, bpw)], idx_v)
        g0 = pltpu.async_copy(table_hbm.at[idx_v.at[pl.ds(0, 128)]],
                              rows_v.at[pl.ds(0, 128)], sem_a)
        g1 = pltpu.async_copy(table_hbm.at[idx_v.at[pl.ds(128, 128)]],
                              rows_v.at[pl.ds(128, 128)], sem_b)
        g0.wait()
        w0 = pltpu.async_copy(rows_v.at[pl.ds(0, 128)],
                              out_hbm.at[pl.ds(base, 128)], sem_w)
        g1.wait()
        w1 = pltpu.async_copy(rows_v.at[pl.ds(128, 128)],
                              out_hbm.at[pl.ds(base + 128, 128)], sem_w)
        w0.wait()
        w1.wait()

    return k(emb, x)


def _project(h, W1, b1, W2, b2, bm):
    n_rows, d_in = h.shape
    d_out = W1.shape[1]

    def body(h_ref, w1_ref, b1_ref, w2_ref, b2_ref, m_ref, n_ref):
        hb = h_ref[...]
        m_ref[...] = jnp.dot(hb, w1_ref[...],
                             preferred_element_type=jnp.float32) + b1_ref[...]
        n_ref[...] = jnp.dot(hb, w2_ref[...],
                             preferred_element_type=jnp.float32) + b2_ref[...]

    return pl.pallas_call(
        body,
        grid=(n_rows // bm,),
        in_specs=[
            pl.BlockSpec((bm, d_in), lambda i: (i, 0)),
            pl.BlockSpec((d_in, d_out), lambda i: (0, 0)),
            pl.BlockSpec((1, d_out), lambda i: (0, 0)),
            pl.BlockSpec((d_in, d_out), lambda i: (0, 0)),
            pl.BlockSpec((1, d_out), lambda i: (0, 0)),
        ],
        out_specs=[
            pl.BlockSpec((bm, d_out), lambda i: (i, 0)),
            pl.BlockSpec((bm, d_out), lambda i: (i, 0)),
        ],
        out_shape=[jax.ShapeDtypeStruct((n_rows, d_out), jnp.float32)] * 2,
    )(h, W1, b1, W2, b2)


def _seg_softmax_pool(h, aw, ab, au, seg_len):
    bm = h.shape[0]
    ns = bm // seg_len
    t = jnp.tanh(jnp.dot(h, aw, preferred_element_type=jnp.float32) + ab)
    a = jnp.sum(t * au, axis=1, keepdims=True)
    e = jnp.exp(a - jnp.max(a))
    rows = lax.broadcasted_iota(jnp.int32, (ns, bm), 0)
    cols = lax.broadcasted_iota(jnp.int32, (ns, bm), 1)
    seg = jnp.where(cols // seg_len == rows, 1.0, 0.0)
    ssum = jnp.dot(seg, e, preferred_element_type=jnp.float32)
    pw = jnp.dot(seg, e * h, preferred_element_type=jnp.float32)
    return pw / ssum


def _fused_mp(adj, m0, n0, attWs, attbs, attus, w1n, b1n, w2n, b2n,
              bn_g, bn_b, fc1_W, fc1_b, attS_W, attS_b, attS_u,
              fc2_W, fc2_b, fc3_W, fc3_b, bm):
    n_rows = adj.shape[0]
    NI = n_rows // bm
    ns = bm // S0
    n_sent = n_rows // S0
    n_doc = n_sent // S1
    nc = fc3_W.shape[1]
    pk = n_rows // NSPLIT
    full = lambda *shape: pl.BlockSpec(shape, lambda l, i: (0,) * len(shape))

    def body(a0, a1, a2, a3, a4, a5, a6, a7, m_ref, n_ref, aw_ref, ab_ref, au_ref,
             w1_ref, b1_ref, w2_ref, b2_ref,
             g_ref, bb_ref, w1h_ref, b1h_ref, asw_ref, asb_ref, asu_ref,
             w2h_ref, b2h_ref, w3h_ref, b3h_ref,
             out_ref, m1_s, n1_s, pool_s):
        l, i = pl.program_id(0), pl.program_id(1)
        adj_refs = (a0, a1, a2, a3, a4, a5, a6, a7)

        @pl.when(l == 0)
        def _():
            acc = sum(
                jnp.dot(adj_refs[j][...], m_ref[pl.ds(j * pk, pk), :],
                        preferred_element_type=jnp.float32)
                for j in range(NSPLIT))
            h = jnp.maximum(acc + n_ref[...], 0.0)
            pool_s[pl.ds(i * ns, ns), :] = _seg_softmax_pool(
                h, aw_ref[0], ab_ref[0], au_ref[0], S0)
            m1_s[pl.ds(i * bm, bm), :] = jnp.dot(
                h, w1_ref[...], preferred_element_type=jnp.float32) + b1_ref[...]
            n1_s[pl.ds(i * bm, bm), :] = jnp.dot(
                h, w2_ref[...], preferred_element_type=jnp.float32) + b2_ref[...]

        @pl.when(l == 1)
        def _():
            acc = sum(
                jnp.dot(adj_refs[j][...], m1_s[pl.ds(j * pk, pk), :],
                        preferred_element_type=jnp.float32)
                for j in range(NSPLIT))
            h = jnp.maximum(acc + n1_s[pl.ds(i * bm, bm), :], 0.0)
            pool_s[pl.ds(n_sent + i * ns, ns), :] = _seg_softmax_pool(
                h, aw_ref[0], ab_ref[0], au_ref[0], S0)

        @pl.when((l == 1) & (i == NI - 1))
        def _():
            z0 = pool_s[pl.ds(0, n_sent), :]
            z1 = pool_s[pl.ds(n_sent, n_sent), :]
            mu0 = jnp.mean(z0, axis=0, keepdims=True)
            v0 = jnp.mean((z0 - mu0) ** 2, axis=0, keepdims=True)
            z0 = ((z0 - mu0) * lax.rsqrt(v0 + 1e-5) * g_ref[:, pl.ds(0, NH)]
                  + bb_ref[:, pl.ds(0, NH)])
            mu1 = jnp.mean(z1, axis=0, keepdims=True)
            v1 = jnp.mean((z1 - mu1) ** 2, axis=0, keepdims=True)
            z1 = ((z1 - mu1) * lax.rsqrt(v1 + 1e-5) * g_ref[:, pl.ds(NH, NH)]
                  + bb_ref[:, pl.ds(NH, NH)])
            zf = jnp.maximum(
                jnp.dot(z0, w1h_ref[pl.ds(0, NH), :],
                        preferred_element_type=jnp.float32)
                + jnp.dot(z1, w1h_ref[pl.ds(NH, NH), :],
                          preferred_element_type=jnp.float32)
                + b1h_ref[...], 0.0)
            zs = _seg_softmax_pool(zf, asw_ref[...], asb_ref[...],
                                   asu_ref[...], S1)
            z2 = jnp.maximum(
                jnp.dot(zs, w2h_ref[...], preferred_element_type=jnp.float32)
                + b2h_ref[...], 0.0)
            z3 = jnp.dot(z2, w3h_ref[...],
                         preferred_element_type=jnp.float32) + b3h_ref[...]
            mx = jnp.max(z3, axis=1, keepdims=True)
            lse = jnp.log(jnp.sum(jnp.exp(z3 - mx), axis=1, keepdims=True))
            out_ref[...] = z3 - mx - lse

    in_specs = (
        [pl.BlockSpec((bm, pk), lambda l, i, j=j: (i, j))
         for j in range(NSPLIT)]
        + [pl.BlockSpec((n_rows, NH), lambda l, i: (0, 0)),
           pl.BlockSpec((bm, NH), lambda l, i: ((1 - l) * i, 0)),
           pl.BlockSpec((1, NH, NH), lambda l, i: (l, 0, 0)),
           pl.BlockSpec((1, 1, NH), lambda l, i: (l, 0, 0)),
           pl.BlockSpec((1, 1, NH), lambda l, i: (l, 0, 0)),
           full(NH, NH), full(1, NH), full(NH, NH), full(1, NH),
           full(1, 2 * NH), full(1, 2 * NH),
           full(2 * NH, NH), full(1, NH),
           full(NH, NH), full(1, NH), full(1, NH),
           full(NH, NH), full(1, NH), full(NH, nc), full(1, nc)])

    return pl.pallas_call(
        body,
        grid=(2, NI),
        in_specs=in_specs,
        out_specs=full(n_doc, nc),
        out_shape=jax.ShapeDtypeStruct((n_doc, nc), jnp.float32),
        scratch_shapes=[
            pltpu.VMEM((n_rows, NH), jnp.float32),
            pltpu.VMEM((n_rows, NH), jnp.float32),
            pltpu.VMEM((2 * n_sent, NH), jnp.float32),
        ],
        compiler_params=pltpu.CompilerParams(
            dimension_semantics=("arbitrary", "arbitrary")),
    )(adj, adj, adj, adj, adj, adj, adj, adj, m0, n0, attWs, attbs, attus,
      w1n, b1n, w2n, b2n, bn_g, bn_b, fc1_W, fc1_b,
      attS_W, attS_b, attS_u, fc2_W, fc2_b, fc3_W, fc3_b)


def kernel(x, adj, adj_s, shapes, emb, params):
    h = _gather_rows(emb, x)
    m0, n0 = _project(
        h,
        params['mp0_W1'], params['mp0_b1'].reshape(1, NH),
        params['mp0_W2'], params['mp0_b2'].reshape(1, NH),
        bm=1024)
    attWs = jnp.stack([params['att0_W'], params['att1_W']])
    attbs = jnp.stack([params['att0_b'].reshape(1, NH),
                       params['att1_b'].reshape(1, NH)])
    attus = jnp.stack([params['att0_u'].reshape(1, NH),
                       params['att1_u'].reshape(1, NH)])
    return _fused_mp(
        adj, m0, n0, attWs, attbs, attus,
        params['mp1_W1'], params['mp1_b1'].reshape(1, NH),
        params['mp1_W2'], params['mp1_b2'].reshape(1, NH),
        params['bn_g'].reshape(1, 2 * NH), params['bn_b'].reshape(1, 2 * NH),
        params['fc1_W'], params['fc1_b'].reshape(1, NH),
        params['attS_W'], params['attS_b'].reshape(1, NH),
        params['attS_u'].reshape(1, NH),
        params['fc2_W'], params['fc2_b'].reshape(1, NH),
        params['fc3_W'], params['fc3_b'].reshape(1, -1),
        bm=512)


# speedup vs baseline: 1.0022x; 1.0022x over previous
import functools

import jax
import jax.numpy as jnp
from jax import lax
from jax.experimental import pallas as pl
from jax.experimental.pallas import tpu as pltpu
from jax.experimental.pallas import tpu_sc as plsc

S0 = 32
S1 = 8
NH = 64
NSPLIT = 4


def _gather_rows(emb, x):
    V, D = emb.shape
    B = x.shape[0]
    NC, NS = 2, 16
    NW = NC * NS
    bpw = B // NW

    mesh = plsc.VectorSubcoreMesh(core_axis_name="c", subcore_axis_name="s")

    @functools.partial(
        pl.kernel,
        mesh=mesh,
        out_type=jax.ShapeDtypeStruct((B, D), jnp.float32),
        scratch_types=[
            pltpu.VMEM((bpw,), jnp.int32),
            pltpu.VMEM((bpw, D), jnp.float32),
            pltpu.SemaphoreType.DMA,
            pltpu.SemaphoreType.DMA,
            pltpu.SemaphoreType.DMA,
        ],
    )
    def k(table_hbm, idx_hbm, out_hbm, idx_v, rows_v, sem_a, sem_b, sem_w):
        wid = lax.axis_index("s") * NC + lax.axis_index("c")
        base = wid * bpw
        pltpu.sync_copy(idx_hbm.at[pl.ds(base, bpw)], idx_v)
        g0 = pltpu.async_copy(table_hbm.at[idx_v.at[pl.ds(0, 128)]],
                              rows_v.at[pl.ds(0, 128)], sem_a)
        g1 = pltpu.async_copy(table_hbm.at[idx_v.at[pl.ds(128, 128)]],
                              rows_v.at[pl.ds(128, 128)], sem_b)
        g0.wait()
        w0 = pltpu.async_copy(rows_v.at[pl.ds(0, 128)],
                              out_hbm.at[pl.ds(base, 128)], sem_w)
        g1.wait()
        w1 = pltpu.async_copy(rows_v.at[pl.ds(128, 128)],
                              out_hbm.at[pl.ds(base + 128, 128)], sem_w)
        w0.wait()
        w1.wait()

    return k(emb, x)


def _seg_softmax_pool(h, aw, ab, au, seg_len):
    bm = h.shape[0]
    ns = bm // seg_len
    t = jnp.tanh(jnp.dot(h, aw, preferred_element_type=jnp.float32) + ab)
    a = jnp.sum(t * au, axis=1, keepdims=True)
    e = jnp.exp(a - jnp.max(a))
    rows = lax.broadcasted_iota(jnp.int32, (ns, bm), 0)
    cols = lax.broadcasted_iota(jnp.int32, (ns, bm), 1)
    seg = jnp.where(cols // seg_len == rows, 1.0, 0.0)
    ssum = jnp.dot(seg, e, preferred_element_type=jnp.float32)
    pw = jnp.dot(seg, e * h, preferred_element_type=jnp.float32)
    return pw / ssum


def _fused_all(adj, h, w10, b10, w20, b20, attWs, attbs, attus,
               w1n, b1n, w2n, b2n,
               bn_g, bn_b, fc1_W, fc1_b, attS_W, attS_b, attS_u,
               fc2_W, fc2_b, fc3_W, fc3_b, bm):
    n_rows = adj.shape[0]
    d_in = h.shape[1]
    NI = n_rows // bm
    ns = bm // S0
    n_sent = n_rows // S0
    n_doc = n_sent // S1
    nc = fc3_W.shape[1]
    pk = n_rows // NSPLIT
    full = lambda *shape: pl.BlockSpec(shape, lambda p, i: (0,) * len(shape))

    def body(a0, a1, a2, a3, h_ref, w10_ref, b10_ref, w20_ref, b20_ref,
             aw_ref, ab_ref, au_ref, w1_ref, b1_ref, w2_ref, b2_ref,
             g_ref, bb_ref, w1h_ref, b1h_ref, asw_ref, asb_ref, asu_ref,
             w2h_ref, b2h_ref, w3h_ref, b3h_ref,
             out_ref, m0_s, n0_s, m1_s, n1_s, pool_s):
        p, i = pl.program_id(0), pl.program_id(1)
        adj_refs = (a0, a1, a2, a3)

        @pl.when(p == 0)
        def _():
            hb = h_ref[...]
            m0_s[pl.ds(i * bm, bm), :] = jnp.dot(
                hb, w10_ref[...], preferred_element_type=jnp.float32) + b10_ref[...]
            n0_s[pl.ds(i * bm, bm), :] = jnp.dot(
                hb, w20_ref[...], preferred_element_type=jnp.float32) + b20_ref[...]

        @pl.when(p == 1)
        def _():
            acc = sum(
                jnp.dot(adj_refs[j][...], m0_s[pl.ds(j * pk, pk), :],
                        preferred_element_type=jnp.float32)
                for j in range(NSPLIT))
            hh = jnp.maximum(acc + n0_s[pl.ds(i * bm, bm), :], 0.0)
            pool_s[pl.ds(i * ns, ns), :] = _seg_softmax_pool(
                hh, aw_ref[0], ab_ref[0], au_ref[0], S0)
            m1_s[pl.ds(i * bm, bm), :] = jnp.dot(
                hh, w1_ref[...], preferred_element_type=jnp.float32) + b1_ref[...]
            n1_s[pl.ds(i * bm, bm), :] = jnp.dot(
                hh, w2_ref[...], preferred_element_type=jnp.float32) + b2_ref[...]

        @pl.when(p == 2)
        def _():
            acc = sum(
                jnp.dot(adj_refs[j][...], m1_s[pl.ds(j * pk, pk), :],
                        preferred_element_type=jnp.float32)
                for j in range(NSPLIT))
            hh = jnp.maximum(acc + n1_s[pl.ds(i * bm, bm), :], 0.0)
            pool_s[pl.ds(n_sent + i * ns, ns), :] = _seg_softmax_pool(
                hh, aw_ref[0], ab_ref[0], au_ref[0], S0)

        @pl.when((p == 2) & (i == NI - 1))
        def _():
            z0 = pool_s[pl.ds(0, n_sent), :]
            z1 = pool_s[pl.ds(n_sent, n_sent), :]
            mu0 = jnp.mean(z0, axis=0, keepdims=True)
            v0 = jnp.mean((z0 - mu0) ** 2, axis=0, keepdims=True)
            z0 = ((z0 - mu0) * lax.rsqrt(v0 + 1e-5) * g_ref[:, pl.ds(0, NH)]
                  + bb_ref[:, pl.ds(0, NH)])
            mu1 = jnp.mean(z1, axis=0, keepdims=True)
            v1 = jnp.mean((z1 - mu1) ** 2, axis=0, keepdims=True)
            z1 = ((z1 - mu1) * lax.rsqrt(v1 + 1e-5) * g_ref[:, pl.ds(NH, NH)]
                  + bb_ref[:, pl.ds(NH, NH)])
            zf = jnp.maximum(
                jnp.dot(z0, w1h_ref[pl.ds(0, NH), :],
                        preferred_element_type=jnp.float32)
                + jnp.dot(z1, w1h_ref[pl.ds(NH, NH), :],
                          preferred_element_type=jnp.float32)
                + b1h_ref[...], 0.0)
            zs = _seg_softmax_pool(zf, asw_ref[...], asb_ref[...],
                                   asu_ref[...], S1)
            z2 = jnp.maximum(
                jnp.dot(zs, w2h_ref[...], preferred_element_type=jnp.float32)
                + b2h_ref[...], 0.0)
            z3 = jnp.dot(z2, w3h_ref[...],
                         preferred_element_type=jnp.float32) + b3h_ref[...]
            mx = jnp.max(z3, axis=1, keepdims=True)
            lse = jnp.log(jnp.sum(jnp.exp(z3 - mx), axis=1, keepdims=True))
            out_ref[...] = z3 - mx - lse

    in_specs = (
        [pl.BlockSpec((bm, pk),
                      lambda p, i, j=j: (jnp.where(p == 0, 0, i), j))
         for j in range(NSPLIT)]
        + [pl.BlockSpec((bm, d_in),
                        lambda p, i: (jnp.where(p == 0, i, NI - 1), 0)),
           full(d_in, NH), full(1, NH), full(d_in, NH), full(1, NH),
           pl.BlockSpec((1, NH, NH),
                        lambda p, i: (jnp.maximum(p, 1) - 1, 0, 0)),
           pl.BlockSpec((1, 1, NH),
                        lambda p, i: (jnp.maximum(p, 1) - 1, 0, 0)),
           pl.BlockSpec((1, 1, NH),
                        lambda p, i: (jnp.maximum(p, 1) - 1, 0, 0)),
           full(NH, NH), full(1, NH), full(NH, NH), full(1, NH),
           full(1, 2 * NH), full(1, 2 * NH),
           full(2 * NH, NH), full(1, NH),
           full(NH, NH), full(1, NH), full(1, NH),
           full(NH, NH), full(1, NH), full(NH, nc), full(1, nc)])

    return pl.pallas_call(
        body,
        grid=(3, NI),
        in_specs=in_specs,
        out_specs=full(n_doc, nc),
        out_shape=jax.ShapeDtypeStruct((n_doc, nc), jnp.float32),
        scratch_shapes=[
            pltpu.VMEM((n_rows, NH), jnp.float32),
            pltpu.VMEM((n_rows, NH), jnp.float32),
            pltpu.VMEM((n_rows, NH), jnp.float32),
            pltpu.VMEM((n_rows, NH), jnp.float32),
            pltpu.VMEM((2 * n_sent, NH), jnp.float32),
        ],
        compiler_params=pltpu.CompilerParams(
            dimension_semantics=("arbitrary", "arbitrary")),
    )(adj, adj, adj, adj, h, w10, b10, w20, b20, attWs, attbs, attus,
      w1n, b1n, w2n, b2n, bn_g, bn_b, fc1_W, fc1_b,
      attS_W, attS_b, attS_u, fc2_W, fc2_b, fc3_W, fc3_b)


def kernel(x, adj, adj_s, shapes, emb, params):
    h = _gather_rows(emb, x)
    attWs = jnp.stack([params['att0_W'], params['att1_W']])
    attbs = jnp.stack([params['att0_b'].reshape(1, NH),
                       params['att1_b'].reshape(1, NH)])
    attus = jnp.stack([params['att0_u'].reshape(1, NH),
                       params['att1_u'].reshape(1, NH)])
    return _fused_all(
        adj, h,
        params['mp0_W1'], params['mp0_b1'].reshape(1, NH),
        params['mp0_W2'], params['mp0_b2'].reshape(1, NH),
        attWs, attbs, attus,
        params['mp1_W1'], params['mp1_b1'].reshape(1, NH),
        params['mp1_W2'], params['mp1_b2'].reshape(1, NH),
        params['bn_g'].reshape(1, 2 * NH), params['bn_b'].reshape(1, 2 * NH),
        params['fc1_W'], params['fc1_b'].reshape(1, NH),
        params['attS_W'], params['attS_b'].reshape(1, NH),
        params['attS_u'].reshape(1, NH),
        params['fc2_W'], params['fc2_b'].reshape(1, NH),
        params['fc3_W'], params['fc3_b'].reshape(1, -1),
        bm=512)
